# baseline (speedup 1.0000x reference)
import functools

import jax
import jax.numpy as jnp
from jax import lax
from jax.experimental import pallas as pl
from jax.experimental.pallas import tpu as pltpu
from jax.experimental.pallas import tpu_sc as plsc

L = 16
NPASS = 2


@functools.lru_cache(maxsize=None)
def _make_kernel(B, ZD, V, D, NC, NS):
    NW = NC * NS
    assert B % (8 * NW) == 0 and V % 2 == 0 and 2 * D == 128
    b_per_w = B // NW
    assert b_per_w % (NPASS * L) == 0 and b_per_w <= 128
    ph = b_per_w // NPASS

    mesh = plsc.VectorSubcoreMesh(
        core_axis_name="c", subcore_axis_name="s", num_cores=NC, num_subcores=NS
    )

    @functools.partial(
        pl.kernel,
        mesh=mesh,
        out_type=jax.ShapeDtypeStruct((B, D), jnp.float32),
        scratch_types=[
            pltpu.VMEM((b_per_w,), jnp.int32),
            pltpu.VMEM((ZD, b_per_w), jnp.int32),
            pltpu.VMEM((NPASS, ph), jnp.int32),
            pltpu.VMEM((b_per_w,), jnp.int32),
            pltpu.VMEM((ph, 2 * D), jnp.float32),
            pltpu.VMEM((ph, 2 * D), jnp.float32),
            pltpu.VMEM((b_per_w, D), jnp.float32),
            pltpu.SemaphoreType.DMA,
            pltpu.SemaphoreType.DMA,
        ],
        compiler_params=pltpu.CompilerParams(needs_layout_passes=False),
    )
    def k(obs_hbm, zt_hbm, table2_hbm, out_hbm,
          obs_v, z_v, idx_v, off_v, pairs0, pairs1, rows_v, sem0, sem1):
        pairs = (pairs0, pairs1)
        sems = (sem0, sem1)
        wid = lax.axis_index("s") * NC + lax.axis_index("c")
        base = wid * b_per_w
        pltpu.sync_copy(obs_hbm.at[pl.ds(base, b_per_w)], obs_v)
        pltpu.sync_copy(zt_hbm.at[:, pl.ds(base, b_per_w)], z_v)
        for g in range(b_per_w // L):
            acc = obs_v[pl.ds(g * L, L)]
            for j in range(ZD):
                bits = z_v[j, pl.ds(g * L, L)]
                acc = acc + bits * (1 << (ZD - 1 - j))
            s = jnp.where(acc >= V, acc - V, acc)
            idx_v[g // (ph // L), pl.ds((g % (ph // L)) * L, L)] = s >> 1
            off_v[pl.ds(g * L, L)] = (s & 1) * D

        cps = [
            pltpu.async_copy(table2_hbm.at[idx_v.at[p]], pairs[p], sems[p])
            for p in range(NPASS)
        ]
        iota = lax.iota(jnp.int32, L)
        for p in range(NPASS):
            cps[p].wait()
            buf = pairs[p]

            def select(col, carry, p=p, buf=buf):
                colv = jnp.zeros((L,), jnp.int32) + col
                for g in range(ph // L):
                    ei = p * ph + g * L
                    w = plsc.load_gather(
                        buf, [iota + g * L, off_v[pl.ds(ei, L)] + colv]
                    )
                    plsc.store_scatter(rows_v, [iota + ei, colv], w)
                return carry

            lax.fori_loop(0, D, select, 0)

        pltpu.sync_copy(rows_v, out_hbm.at[pl.ds(base, b_per_w)])

    return k


def kernel(obs_hash, z, seed_to_action):
    B, ZD = z.shape
    V, D = seed_to_action.shape
    info = plsc.get_sparse_core_info()
    k = _make_kernel(B, ZD, V, D, info.num_cores, info.num_subcores)
    return k(
        obs_hash.astype(jnp.int32),
        z.astype(jnp.int32).T,
        seed_to_action.reshape(V // 2, 2 * D),
    )


# speedup vs baseline: 1.0699x; 1.0699x over previous
import functools

import jax
import jax.numpy as jnp
from jax import lax
from jax.experimental import pallas as pl
from jax.experimental.pallas import tpu as pltpu
from jax.experimental.pallas import tpu_sc as plsc

L = 16
NPASS = 2


@functools.lru_cache(maxsize=None)
def _make_kernel(B, ZD, V, D, NC, NS):
    NW = NC * NS
    assert B % (8 * NW) == 0 and V % 2 == 0 and 2 * D == 128
    b_per_w = B // NW
    assert b_per_w % (NPASS * L) == 0 and b_per_w <= 128
    ph = b_per_w // NPASS

    mesh = plsc.VectorSubcoreMesh(
        core_axis_name="c", subcore_axis_name="s", num_cores=NC, num_subcores=NS
    )

    @functools.partial(
        pl.kernel,
        mesh=mesh,
        out_type=jax.ShapeDtypeStruct((B, D), jnp.float32),
        scratch_types=[
            pltpu.VMEM((b_per_w,), jnp.int32),
            pltpu.VMEM((b_per_w, ZD), jnp.int32),
            pltpu.VMEM((NPASS, ph), jnp.int32),
            pltpu.VMEM((b_per_w,), jnp.int32),
            pltpu.VMEM((ph, 2 * D), jnp.float32),
            pltpu.VMEM((ph, 2 * D), jnp.float32),
            pltpu.VMEM((b_per_w, D), jnp.float32),
            pltpu.SemaphoreType.DMA,
            pltpu.SemaphoreType.DMA,
        ],
        compiler_params=pltpu.CompilerParams(needs_layout_passes=False),
    )
    def k(obs_hbm, z_hbm, table2_hbm, out_hbm,
          obs_v, z_v, idx_v, off_v, pairs0, pairs1, rows_v, sem0, sem1):
        pairs = (pairs0, pairs1)
        sems = (sem0, sem1)
        wid = lax.axis_index("s") * NC + lax.axis_index("c")
        base = wid * b_per_w
        pltpu.sync_copy(obs_hbm.at[pl.ds(base, b_per_w)], obs_v)
        pltpu.sync_copy(z_hbm.at[pl.ds(base, b_per_w)], z_v)
        iota = lax.iota(jnp.int32, L)
        for g in range(b_per_w // L):
            acc = obs_v[pl.ds(g * L, L)]
            for j in range(ZD):
                bits = plsc.load_gather(
                    z_v, [iota + g * L, jnp.full((L,), j, jnp.int32)]
                )
                acc = acc + bits * (1 << (ZD - 1 - j))
            s = jnp.where(acc >= V, acc - V, acc)
            idx_v[g // (ph // L), pl.ds((g % (ph // L)) * L, L)] = s >> 1
            off_v[pl.ds(g * L, L)] = (s & 1) * D

        cps = [
            pltpu.async_copy(table2_hbm.at[idx_v.at[p]], pairs[p], sems[p])
            for p in range(NPASS)
        ]
        for p in range(NPASS):
            cps[p].wait()
            buf = pairs[p]

            @plsc.parallel_loop(0, D, 1, unroll=4)
            def select(col, p=p, buf=buf):
                colv = jnp.zeros((L,), jnp.int32) + col
                for g in range(ph // L):
                    ei = p * ph + g * L
                    w = plsc.load_gather(
                        buf, [iota + g * L, off_v[pl.ds(ei, L)] + colv]
                    )
                    plsc.store_scatter(rows_v, [iota + ei, colv], w)

        pltpu.sync_copy(rows_v, out_hbm.at[pl.ds(base, b_per_w)])

    return k


def kernel(obs_hash, z, seed_to_action):
    B, ZD = z.shape
    V, D = seed_to_action.shape
    info = plsc.get_sparse_core_info()
    k = _make_kernel(B, ZD, V, D, info.num_cores, info.num_subcores)
    return k(
        obs_hash.astype(jnp.int32),
        z.astype(jnp.int32),
        seed_to_action.reshape(V // 2, 2 * D),
    )
